# baseline (speedup 1.0000x reference)
import functools

import jax
import jax.numpy as jnp
from jax import lax
from jax.experimental import pallas as pl
from jax.experimental.pallas import tpu as pltpu
from jax.experimental.pallas import tpu_sc as plsc

N = 100000
D = 128
B = 64
R = 10000
G = N // R
NW = 32
CH = 80
NCH = N // CH
CW = 16
ND = D // 16


def _sc_stats(x, seg2):
    mesh = plsc.VectorSubcoreMesh(core_axis_name="c", subcore_axis_name="s")

    @functools.partial(
        pl.kernel, mesh=mesh,
        out_type=[
            jax.ShapeDtypeStruct((NW, B, D), jnp.float32),
            jax.ShapeDtypeStruct((NW, B, D), jnp.float32),
            jax.ShapeDtypeStruct((NW, B, CW), jnp.float32),
        ],
        scratch_types=[
            pltpu.VMEM((CH, D), jnp.float32),
            pltpu.VMEM((144,), jnp.int32),
            pltpu.VMEM((B, D), jnp.float32),
            pltpu.VMEM((B, D), jnp.float32),
            pltpu.VMEM((B, CW), jnp.float32),
        ],
    )
    def k(x_hbm, seg_hbm, sum_out, sq_out, cnt_out,
          xbuf, idxbuf, accs, accq, cntb):
        c = lax.axis_index("c")
        s = lax.axis_index("s")
        wid = s * 2 + c

        zero = jnp.zeros((16,), jnp.float32)

        def zrow(b, _):
            for d in range(ND):
                accs[b, pl.ds(16 * d, 16)] = zero
                accq[b, pl.ds(16 * d, 16)] = zero
            cntb[b] = zero
            return 0
        lax.fori_loop(0, B, zrow, 0)

        zcarry = (jnp.int32(0), jnp.int32(0)) + tuple([zero] * (2 * ND))

        def flush(carry):
            sp = carry[0]
            rl = carry[1]

            def do(_):
                for d in range(ND):
                    accs[sp, pl.ds(16 * d, 16)] += carry[2 + d]
                    accq[sp, pl.ds(16 * d, 16)] += carry[2 + ND + d]
                cntb[sp] += jnp.full((16,), rl, jnp.float32)
                return 0
            lax.cond(rl > 0, do, lambda _: 0, 0)

        def row_body(r, carry):
            sp = carry[0]
            sv = idxbuf[pl.ds(r, 16)][0]

            def flush_branch(cy):
                flush(cy)
                return (sv, jnp.int32(0)) + tuple([zero] * (2 * ND))

            carry = lax.cond(sv != sp, flush_branch, lambda cy: cy, carry)
            rl = carry[1]
            vals = []
            sqs = []
            for d in range(ND):
                v = xbuf[r, pl.ds(16 * d, 16)]
                vals.append(carry[2 + d] + v)
                sqs.append(carry[2 + ND + d] + v * v)
            return (sv, rl + 1) + tuple(vals) + tuple(sqs)

        def chunk(t, carry):
            kk = wid + t * NW
            pltpu.sync_copy(x_hbm.at[pl.ds(kk * CH, CH)], xbuf)
            pltpu.sync_copy(seg_hbm.at[kk], idxbuf.at[pl.ds(0, 128)])
            return lax.fori_loop(0, CH, row_body, carry)

        nt = (NCH - wid - 1) // NW + 1
        carry = lax.fori_loop(0, nt, chunk, zcarry)
        flush(carry)

        pltpu.sync_copy(accs, sum_out.at[wid])
        pltpu.sync_copy(accq, sq_out.at[wid])
        pltpu.sync_copy(cntb, cnt_out.at[wid])

    return k(x, seg2)


def _apply_body(ids_ref, x_ref, sum_ref, sq_ref, cnt_ref,
                w_ref, b_ref, ms_ref, out_ref, tab_ref):
    i = pl.program_id(0)

    @pl.when(i == 0)
    def _():
        s = jnp.sum(sum_ref[...], axis=0)
        q = jnp.sum(sq_ref[...], axis=0)
        c = jnp.maximum(jnp.sum(cnt_ref[...], axis=0)[:, 0:1], 1.0)
        mean = s / c
        mm = mean * ms_ref[...]
        segsq = q - 2.0 * mm * s + c * mm * mm
        rstd = lax.rsqrt(segsq / c + 1e-6)
        scale = w_ref[...] * rstd
        shift = b_ref[...] - mm * scale
        tab_ref[...] = jnp.concatenate([scale, shift],
                                       axis=1).astype(jnp.bfloat16)

    ids = ids_ref[0]
    iota = lax.broadcasted_iota(jnp.int32, (B, 1), 0)
    oh = (iota == ids).astype(jnp.bfloat16)
    g = lax.dot_general(oh, tab_ref[...], (((0,), (0,)), ((), ())),
                        preferred_element_type=jnp.float32)
    out_ref[...] = x_ref[...] * g[:, :D] + g[:, D:]


def _tc_apply(x, seg_row, sums, sqs, cnts, weight, bias, mean_scale):
    return pl.pallas_call(
        _apply_body,
        grid=(G,),
        in_specs=[
            pl.BlockSpec((1, 1, R), lambda i: (i, 0, 0)),
            pl.BlockSpec((R, D), lambda i: (i, 0)),
            pl.BlockSpec((NW, B, D), lambda i: (0, 0, 0)),
            pl.BlockSpec((NW, B, D), lambda i: (0, 0, 0)),
            pl.BlockSpec((NW, B, CW), lambda i: (0, 0, 0)),
            pl.BlockSpec((1, D), lambda i: (0, 0)),
            pl.BlockSpec((1, D), lambda i: (0, 0)),
            pl.BlockSpec((1, D), lambda i: (0, 0)),
        ],
        out_specs=pl.BlockSpec((R, D), lambda i: (i, 0)),
        out_shape=jax.ShapeDtypeStruct((N, D), jnp.float32),
        scratch_shapes=[pltpu.VMEM((B, 2 * D), jnp.bfloat16)],
    )(seg_row, x, sums, sqs, cnts, weight, bias, mean_scale)


def kernel(x, segment_ids, weight, bias, mean_scale):
    seg = segment_ids.astype(jnp.int32)
    seg_pad = jnp.pad(seg.reshape(NCH, CH), ((0, 0), (0, 128 - CH)))
    sums, sqs, cnts = _sc_stats(x, seg_pad)
    return _tc_apply(x, seg.reshape(G, 1, R), sums, sqs, cnts,
                     weight.reshape(1, D), bias.reshape(1, D),
                     mean_scale.reshape(1, D))


# speedup vs baseline: 1.4294x; 1.4294x over previous
import functools

import jax
import jax.numpy as jnp
from jax import lax
from jax.experimental import pallas as pl
from jax.experimental.pallas import tpu as pltpu
from jax.experimental.pallas import tpu_sc as plsc

N = 100000
D = 128
B = 64
R = 10000
G = N // R
NW = 32
CH = 80
NCH = N // CH
CW = 16
ND = D // 16


def _sc_stats(x, seg2):
    mesh = plsc.VectorSubcoreMesh(core_axis_name="c", subcore_axis_name="s")

    @functools.partial(
        pl.kernel, mesh=mesh,
        out_type=[
            jax.ShapeDtypeStruct((NW, B, D), jnp.float32),
            jax.ShapeDtypeStruct((NW, B, D), jnp.float32),
            jax.ShapeDtypeStruct((NW, B, CW), jnp.float32),
        ],
        scratch_types=[
            pltpu.VMEM((CH, D), jnp.float32),
            pltpu.VMEM((144,), jnp.int32),
            pltpu.VMEM((B, D), jnp.float32),
            pltpu.VMEM((B, D), jnp.float32),
            pltpu.VMEM((B, CW), jnp.float32),
        ],
    )
    def k(x_hbm, seg_hbm, sum_out, sq_out, cnt_out,
          xbuf, idxbuf, accs, accq, cntb):
        c = lax.axis_index("c")
        s = lax.axis_index("s")
        wid = s * 2 + c

        zero = jnp.zeros((16,), jnp.float32)

        def zrow(b, _):
            for d in range(ND):
                accs[b, pl.ds(16 * d, 16)] = zero
                accq[b, pl.ds(16 * d, 16)] = zero
            cntb[b] = zero
            return 0
        lax.fori_loop(0, B, zrow, 0)

        def chunk(t, _):
            kk = wid + t * NW
            pltpu.sync_copy(x_hbm.at[pl.ds(kk * CH, CH)], xbuf)
            pltpu.sync_copy(seg_hbm.at[kk], idxbuf.at[pl.ds(0, 128)])
            sv0 = idxbuf[pl.ds(0, 16)][0]
            svl = idxbuf[pl.ds(CH - 16, 16)][15]

            def fast(_):
                def frow(r, cy):
                    vals = []
                    for d in range(ND):
                        v = xbuf[r, pl.ds(16 * d, 16)]
                        vals.append(cy[d] + v)
                        vals.append(cy[ND + d] + v * v)
                    return tuple(vals[0::2]) + tuple(vals[1::2])

                acc = lax.fori_loop(0, CH, frow, tuple([zero] * (2 * ND)))
                for d in range(ND):
                    accs[sv0, pl.ds(16 * d, 16)] += acc[d]
                    accq[sv0, pl.ds(16 * d, 16)] += acc[ND + d]
                cntb[sv0] += jnp.full((16,), CH, jnp.float32)
                return 0

            def slow(_):
                def srow(r, _c):
                    sv = idxbuf[pl.ds(r, 16)][0]
                    for d in range(ND):
                        v = xbuf[r, pl.ds(16 * d, 16)]
                        accs[sv, pl.ds(16 * d, 16)] += v
                        accq[sv, pl.ds(16 * d, 16)] += v * v
                    cntb[sv] += jnp.full((16,), 1.0, jnp.float32)
                    return 0
                return lax.fori_loop(0, CH, srow, 0)

            lax.cond(sv0 == svl, fast, slow, 0)
            return 0

        nt = (NCH - wid - 1) // NW + 1
        lax.fori_loop(0, nt, chunk, 0)

        pltpu.sync_copy(accs, sum_out.at[wid])
        pltpu.sync_copy(accq, sq_out.at[wid])
        pltpu.sync_copy(cntb, cnt_out.at[wid])

    return k(x, seg2)


def _apply_body(ids_ref, x_ref, sum_ref, sq_ref, cnt_ref,
                w_ref, b_ref, ms_ref, out_ref, tab_ref):
    i = pl.program_id(0)

    @pl.when(i == 0)
    def _():
        s = jnp.sum(sum_ref[...], axis=0)
        q = jnp.sum(sq_ref[...], axis=0)
        c = jnp.maximum(jnp.sum(cnt_ref[...], axis=0)[:, 0:1], 1.0)
        mean = s / c
        mm = mean * ms_ref[...]
        segsq = q - 2.0 * mm * s + c * mm * mm
        rstd = lax.rsqrt(segsq / c + 1e-6)
        scale = w_ref[...] * rstd
        shift = b_ref[...] - mm * scale
        tab_ref[...] = jnp.concatenate([scale, shift],
                                       axis=1).astype(jnp.bfloat16)

    ids = ids_ref[0]
    iota = lax.broadcasted_iota(jnp.int32, (B, 1), 0)
    oh = (iota == ids).astype(jnp.bfloat16)
    g = lax.dot_general(oh, tab_ref[...], (((0,), (0,)), ((), ())),
                        preferred_element_type=jnp.float32)
    out_ref[...] = x_ref[...] * g[:, :D] + g[:, D:]


def _tc_apply(x, seg_row, sums, sqs, cnts, weight, bias, mean_scale):
    return pl.pallas_call(
        _apply_body,
        grid=(G,),
        in_specs=[
            pl.BlockSpec((1, 1, R), lambda i: (i, 0, 0)),
            pl.BlockSpec((R, D), lambda i: (i, 0)),
            pl.BlockSpec((NW, B, D), lambda i: (0, 0, 0)),
            pl.BlockSpec((NW, B, D), lambda i: (0, 0, 0)),
            pl.BlockSpec((NW, B, CW), lambda i: (0, 0, 0)),
            pl.BlockSpec((1, D), lambda i: (0, 0)),
            pl.BlockSpec((1, D), lambda i: (0, 0)),
            pl.BlockSpec((1, D), lambda i: (0, 0)),
        ],
        out_specs=pl.BlockSpec((R, D), lambda i: (i, 0)),
        out_shape=jax.ShapeDtypeStruct((N, D), jnp.float32),
        scratch_shapes=[pltpu.VMEM((B, 2 * D), jnp.bfloat16)],
    )(seg_row, x, sums, sqs, cnts, weight, bias, mean_scale)


def kernel(x, segment_ids, weight, bias, mean_scale):
    seg = segment_ids.astype(jnp.int32)
    seg_pad = jnp.pad(seg.reshape(NCH, CH), ((0, 0), (0, 128 - CH)))
    sums, sqs, cnts = _sc_stats(x, seg_pad)
    return _tc_apply(x, seg.reshape(G, 1, R), sums, sqs, cnts,
                     weight.reshape(1, D), bias.reshape(1, D),
                     mean_scale.reshape(1, D))
